# baseline (speedup 1.0000x reference)
import functools

import jax
import jax.numpy as jnp
from jax import lax
from jax.experimental import pallas as pl
from jax.experimental.pallas import tpu as pltpu
from jax.experimental.pallas import tpu_sc as plsc

ICO_N_IN = 10242
N_OUT = 40962
C = 256
LANES = 16
GRP = 2 * LANES
CP = C // 2
CH = 64
T_FULL = N_OUT // CH
TAIL = N_OUT - T_FULL * CH
T = T_FULL + 1
NW = 32
MAXK = -(-T // NW)
NWID = T - (MAXK - 1) * NW
PAD_CHUNKS = MAXK * NW


def _build(mesh, scale):
    @functools.partial(
        pl.kernel,
        out_type=jax.ShapeDtypeStruct((N_OUT, C), jnp.float32),
        mesh=mesh,
        scratch_types=[
            pltpu.VMEM((MAXK * CH,), jnp.int32),
            pltpu.VMEM((MAXK * CH,), jnp.int32),
            pltpu.VMEM((CH, CP), jnp.int32),
            pltpu.VMEM((CH, CP), jnp.int32),
            pltpu.VMEM((CH, CP), jnp.int32),
            pltpu.VMEM((CH, CP), jnp.int32),
            pltpu.VMEM((CH, C), jnp.float32),
            pltpu.VMEM((CH, C), jnp.float32),
            pltpu.VMEM((LANES, C), jnp.float32),
            pltpu.VMEM((LANES,), jnp.int32),
            pltpu.SemaphoreType.DMA,
            pltpu.SemaphoreType.DMA,
            pltpu.SemaphoreType.DMA,
            pltpu.SemaphoreType.DMA,
            pltpu.SemaphoreType.DMA,
        ],
    )
    def k(xs_hbm, idx0_hbm, idx1_hbm, out_hbm,
          i0, i1, b0a, b0b, b1a, b1b, ova, ovb, tl, tidx,
          g0, g1, st0, st1, gi):
        w = lax.axis_index("s") * 2 + lax.axis_index("c")
        start = MAXK * w - lax.max(w - NWID, 0)
        n_w = MAXK - (w >= NWID).astype(jnp.int32)

        ci0 = pltpu.async_copy(idx0_hbm.at[pl.ds(start * CH, MAXK * CH)], i0, gi)
        ci1 = pltpu.async_copy(idx1_hbm.at[pl.ds(start * CH, MAXK * CH)], i1, gi)
        ci0.wait()
        ci1.wait()

        b0 = (b0a, b0b)
        b1 = (b1a, b1b)
        ov = (ova, ovb)
        g = (g0, g1)
        st = (st0, st1)
        mhi = jnp.int32(-65536)

        def gather_pair(kk):
            s = kk & 1
            isl = pl.ds(kk * CH, CH)
            return (pltpu.make_async_copy(xs_hbm.at[i0.at[isl]], b0[s], g[s]),
                    pltpu.make_async_copy(xs_hbm.at[i1.at[isl]], b1[s], g[s]))

        def fire_gathers(kk):
            c0, c1 = gather_pair(kk)
            c0.start()
            c1.start()

        def store_copy(kk):
            s = kk & 1
            t = start + kk
            return pltpu.make_async_copy(
                ov[s], out_hbm.at[pl.ds(t * CH, CH)], st[s])

        def avg_row(dst, dst_row, b0s, b1s, row, q):
            gsl = pl.ds(q * LANES, LANES)
            w0 = b0s[row, gsl]
            w1 = b1s[row, gsl]
            lo = (lax.bitcast_convert_type(w0 * jnp.int32(65536), jnp.float32)
                  + lax.bitcast_convert_type(w1 * jnp.int32(65536), jnp.float32))
            hi = (lax.bitcast_convert_type(w0 & mhi, jnp.float32)
                  + lax.bitcast_convert_type(w1 & mhi, jnp.float32))
            dst[dst_row, pl.ds(q * GRP, LANES)] = lo
            dst[dst_row, pl.ds(q * GRP + LANES, LANES)] = hi

        fire_gathers(0)
        for kk in range(MAXK):
            s = kk & 1
            if kk + 1 < MAXK:
                @pl.when(kk + 1 < n_w)
                def _(kk=kk):
                    fire_gathers(kk + 1)

            @pl.when(kk < n_w)
            def _(kk=kk, s=s):
                t = start + kk
                c0, c1 = gather_pair(kk)
                c0.wait()
                c1.wait()

                @pl.when(t < T_FULL)
                def _():
                    if kk >= 2:
                        store_copy(kk - 2).wait()

                    @plsc.parallel_loop(0, CH * (C // GRP), step=1, unroll=8)
                    def _(qq):
                        avg_row(ov[s], qq >> 3, b0[s], b1[s], qq >> 3, qq & 7)

                    store_copy(kk).start()

                @pl.when(t == T_FULL)
                def _():
                    if kk >= 2:
                        store_copy(kk - 2).wait()
                    for r in range(TAIL):
                        for q in range(C // GRP):
                            avg_row(tl, r, b0[s], b1[s], r, q)

                    def rep_body(r, _):
                        for j in range(C // LANES):
                            sl = pl.ds(j * LANES, LANES)
                            tl[r, sl] = tl[r & 1, sl]
                        return 0

                    lax.fori_loop(TAIL, LANES, rep_body, 0)
                    rows = T_FULL * CH + (
                        lax.iota(jnp.int32, LANES) & (TAIL - 1))
                    tidx[...] = rows
                    pltpu.async_copy(tl, out_hbm.at[tidx], gi).wait()

        for kk in range(MAXK):
            @pl.when((kk < n_w) & (kk >= n_w - 2) & (start + kk < T_FULL))
            def _(kk=kk):
                store_copy(kk).wait()

    return k


def kernel(x, batch_size, reverse_hex):
    del batch_size
    rh = reverse_hex.astype(jnp.int32)
    pad = PAD_CHUNKS * CH - N_OUT
    idx0 = jnp.pad(rh[:, 0], (0, pad))
    idx1 = jnp.pad(rh[:, 1], (0, pad))
    scale = 1.0 / (x.shape[0] // ICO_N_IN)
    u = jax.lax.bitcast_convert_type(
        (x[:ICO_N_IN] * jnp.float32(scale)).astype(jnp.float32),
        jnp.uint32).reshape(ICO_N_IN, C // GRP, GRP)
    ulo, uhi = u[:, :, :LANES], u[:, :, LANES:]

    def _rne(v):
        return (v + jnp.uint32(0x7FFF) + ((v >> 16) & jnp.uint32(1))) & jnp.uint32(0xFFFF0000)

    packed = (_rne(ulo) >> 16) | _rne(uhi)
    xs32 = jax.lax.bitcast_convert_type(packed, jnp.int32).reshape(ICO_N_IN, CP)
    mesh = plsc.VectorSubcoreMesh(core_axis_name="c", subcore_axis_name="s")
    return _build(mesh, scale)(xs32, idx0, idx1)


# speedup vs baseline: 1.1900x; 1.1900x over previous
import functools

import jax
import jax.numpy as jnp
from jax import lax
from jax.experimental import pallas as pl
from jax.experimental.pallas import tpu as pltpu
from jax.experimental.pallas import tpu_sc as plsc

ICO_N_IN = 10242
N_OUT = 40962
C = 256
LANES = 16
CH = 64
T_FULL = N_OUT // CH
TAIL = N_OUT - T_FULL * CH
NW = 32
KPW = T_FULL // NW
MAXK = KPW + 1
PAD_CHUNKS = MAXK * NW


def _build(mesh, scale):
    @functools.partial(
        pl.kernel,
        out_type=jax.ShapeDtypeStruct((N_OUT, C), jnp.float32),
        mesh=mesh,
        scratch_types=[
            pltpu.VMEM((MAXK * CH,), jnp.int32),
            pltpu.VMEM((MAXK * CH,), jnp.int32),
            pltpu.VMEM((CH, C), jnp.float32),
            pltpu.VMEM((CH, C), jnp.float32),
            pltpu.VMEM((CH, C), jnp.float32),
            pltpu.VMEM((CH, C), jnp.float32),
            pltpu.VMEM((CH, C), jnp.float32),
            pltpu.VMEM((CH, C), jnp.float32),
            pltpu.VMEM((LANES, C), jnp.float32),
            pltpu.VMEM((LANES,), jnp.int32),
            pltpu.SemaphoreType.DMA,
            pltpu.SemaphoreType.DMA,
            pltpu.SemaphoreType.DMA,
            pltpu.SemaphoreType.DMA,
            pltpu.SemaphoreType.DMA,
        ],
    )
    def k(x_hbm, idx0_hbm, idx1_hbm, out_hbm,
          i0, i1, b0a, b0b, b1a, b1b, ova, ovb, tl, tidx,
          g0, g1, st0, st1, gi):
        w = lax.axis_index("s") * 2 + lax.axis_index("c")
        start = KPW * w
        last = w == NW - 1

        ci0 = pltpu.async_copy(idx0_hbm.at[pl.ds(start * CH, MAXK * CH)], i0, gi)
        ci1 = pltpu.async_copy(idx1_hbm.at[pl.ds(start * CH, MAXK * CH)], i1, gi)
        ci0.wait()
        ci1.wait()

        b0 = (b0a, b0b)
        b1 = (b1a, b1b)
        ov = (ova, ovb)
        g = (g0, g1)
        st = (st0, st1)

        def gather_pair(kk):
            s = kk & 1
            isl = pl.ds(kk * CH, CH)
            return (pltpu.make_async_copy(x_hbm.at[i0.at[isl]], b0[s], g[s]),
                    pltpu.make_async_copy(x_hbm.at[i1.at[isl]], b1[s], g[s]))

        def fire_gathers(kk):
            c0, c1 = gather_pair(kk)
            c0.start()
            c1.start()

        def store_copy(kk):
            s = kk & 1
            return pltpu.make_async_copy(
                ov[s], out_hbm.at[pl.ds((start + kk) * CH, CH)], st[s])

        fire_gathers(0)
        for kk in range(KPW):
            s = kk & 1
            if kk + 1 < KPW:
                fire_gathers(kk + 1)
            else:
                @pl.when(last)
                def _():
                    fire_gathers(KPW)

            c0, c1 = gather_pair(kk)
            c0.wait()
            c1.wait()
            if kk >= 2:
                store_copy(kk - 2).wait()

            @plsc.parallel_loop(0, CH * (C // LANES), step=1, unroll=8)
            def _(q, s=s):
                i = q >> 4
                sl = pl.ds((q & 15) * LANES, LANES)
                ov[s][i, sl] = (b0[s][i, sl] + b1[s][i, sl]) * scale

            store_copy(kk).start()

        @pl.when(last)
        def _():
            s = KPW & 1
            c0, c1 = gather_pair(KPW)
            c0.wait()
            c1.wait()
            for r in range(TAIL):
                for j in range(C // LANES):
                    sl = pl.ds(j * LANES, LANES)
                    tl[r, sl] = (b0[s][r, sl] + b1[s][r, sl]) * scale

            def rep_body(r, _):
                for j in range(C // LANES):
                    sl = pl.ds(j * LANES, LANES)
                    tl[r, sl] = tl[r & 1, sl]
                return 0

            lax.fori_loop(TAIL, LANES, rep_body, 0)
            rows = T_FULL * CH + (lax.iota(jnp.int32, LANES) & (TAIL - 1))
            tidx[...] = rows
            pltpu.async_copy(tl, out_hbm.at[tidx], gi).wait()

        store_copy(KPW - 2).wait()
        store_copy(KPW - 1).wait()

    return k


def kernel(x, batch_size, reverse_hex):
    del batch_size
    rh = reverse_hex.astype(jnp.int32)
    pad = PAD_CHUNKS * CH - N_OUT
    idx0 = jnp.pad(rh[:, 0], (0, pad))
    idx1 = jnp.pad(rh[:, 1], (0, pad))
    scale = 1.0 / (x.shape[0] // ICO_N_IN)
    mesh = plsc.VectorSubcoreMesh(core_axis_name="c", subcore_axis_name="s")
    return _build(mesh, scale)(x, idx0, idx1)
